# Initial kernel scaffold; baseline (speedup 1.0000x reference)
import jax
import jax.numpy as jnp
from jax.experimental import pallas as pl


def kernel(t, x, edge_index, edge_weight, alpha_train, beta_train, source_train):
    raise NotImplementedError("write your pallas kernel here")


# speedup vs baseline: 4.6121x; 4.6121x over previous
import functools

import jax
import jax.numpy as jnp
from jax import lax
from jax.experimental import pallas as pl
from jax.experimental.pallas import tpu as pltpu
from jax.experimental.pallas import tpu_sc as plsc

N = 10000
E = 320000
D = 128

NC = 2
NS = 16
NW = NC * NS
CHUNK = 128
CPT = 79
E_PAD = NW * CPT * CHUNK

N_ACC = 10240
ZROWS = N_ACC // NS
WROWS = 632
N_OUT = NS * WROWS


def _sc_body(x_hbm, col_hbm, row_hbm, w_hbm, out_hbm,
             colv, rowv, wv, rows, accum, sem):
    c = lax.axis_index("c")
    s = lax.axis_index("s")
    wid = s * NC + c

    pltpu.sync_copy(col_hbm.at[wid], colv)
    pltpu.sync_copy(row_hbm.at[wid], rowv)
    pltpu.sync_copy(w_hbm.at[wid], wv)

    zero16 = jnp.zeros((16,), jnp.float32)

    def zrow(i, carry):
        for t in range(D // 16):
            rows[i, pl.ds(t * 16, 16)] = zero16
        return carry

    lax.fori_loop(0, CHUNK, zrow, 0)
    for rep in range(ZROWS // CHUNK):
        pltpu.sync_copy(rows, accum.at[pl.ds(s * ZROWS + rep * CHUNK, CHUNK)])
    plsc.subcore_barrier()

    def chunk_body(j, carry):
        pltpu.async_copy(x_hbm.at[colv.at[j]], rows, sem).wait()
        jbase = j * CHUNK

        def group_body(g, gcarry):
            ebase = g * 16
            wvec = wv[pl.ds(jbase + ebase, 16)]
            for l in range(16):
                w16 = jnp.broadcast_to(wvec[l], (16,))
                e = ebase + l
                for t in range(D // 16):
                    sl = pl.ds(t * 16, 16)
                    rows[e, sl] = rows[e, sl] * w16
            return gcarry

        lax.fori_loop(0, CHUNK // 16, group_body, 0)
        pltpu.sync_copy(rows, accum.at[rowv.at[j]], add=True)
        return carry

    lax.fori_loop(0, CPT, chunk_body, 0)
    plsc.subcore_barrier()

    pltpu.sync_copy(accum.at[pl.ds(s * WROWS, WROWS)],
                    out_hbm.at[c, pl.ds(s * WROWS, WROWS)])


@jax.jit
def _sc_spmm(x, col3, row3, w3):
    mesh = plsc.VectorSubcoreMesh(core_axis_name="c", subcore_axis_name="s")
    return pl.kernel(
        _sc_body,
        mesh=mesh,
        out_type=jax.ShapeDtypeStruct((NC, N_OUT, D), jnp.float32),
        scratch_types=[
            pltpu.VMEM((CPT, CHUNK), jnp.int32),
            pltpu.VMEM((CPT, CHUNK), jnp.int32),
            pltpu.VMEM((CPT * CHUNK,), jnp.float32),
            pltpu.VMEM((CHUNK, D), jnp.float32),
            pltpu.VMEM_SHARED((N_ACC, D), jnp.float32),
            pltpu.SemaphoreType.DMA,
        ],
    )(x, col3, row3, w3)


def _fin_body(a_ref, c1_ref, b_ref, p_ref, x_ref, o_ref):
    ax = p_ref[0] + p_ref[1]
    xv = x_ref[...]
    o_ref[...] = a_ref[0, 0] * ax + c1_ref[0, 0] * xv - b_ref[0, 0] * (xv * xv)


BR = 2000


@jax.jit
def _finalize(p, x, alpha, c1, beta):
    sspec = pl.BlockSpec(memory_space=pltpu.SMEM)
    return pl.pallas_call(
        _fin_body,
        grid=(N // BR,),
        in_specs=[
            sspec,
            sspec,
            sspec,
            pl.BlockSpec((NC, BR, D), lambda i: (0, i, 0)),
            pl.BlockSpec((BR, D), lambda i: (i, 0)),
        ],
        out_specs=pl.BlockSpec((BR, D), lambda i: (i, 0)),
        out_shape=jax.ShapeDtypeStruct((N, D), jnp.float32),
    )(alpha.reshape(1, 1), c1.reshape(1, 1), beta.reshape(1, 1), p, x)


def kernel(t, x, edge_index, edge_weight, alpha_train, beta_train, source_train):
    row = edge_index[0]
    col = edge_index[1]
    pad = E_PAD - E
    col3 = jnp.concatenate([col, jnp.zeros((pad,), jnp.int32)]).reshape(
        NW, CPT, CHUNK)
    row3 = jnp.concatenate([row, jnp.zeros((pad,), jnp.int32)]).reshape(
        NW, CPT, CHUNK)
    w3 = jnp.concatenate(
        [edge_weight, jnp.zeros((pad,), jnp.float32)]).reshape(NW, CPT * CHUNK)

    partials = _sc_spmm(x, col3, row3, w3)

    alpha = jax.nn.sigmoid(alpha_train) * 0.1
    beta = jax.nn.sigmoid(beta_train) * 0.1
    c1 = beta - alpha + 0.1 * source_train
    return _finalize(partials, x, alpha.astype(jnp.float32),
                     c1.astype(jnp.float32), beta.astype(jnp.float32))
